# baseline (speedup 1.0000x reference)
import functools

import jax
import jax.numpy as jnp
import numpy as np
from jax import lax
from jax.experimental import pallas as pl
from jax.experimental.pallas import tpu as pltpu
from jax.experimental.pallas import tpu_sc as plsc

N = 10000
E = 320000
M = 3
IN = 128
H = 8
OUT = 64
HO = H * OUT
B = 4096

NC = 2
NS = 16
L = 16

ROWS_TC = 1000
EPT = E // NS
SEG = 2000
NSEG = EPT // SEG
PCAP = SEG + 64
CK = 64
NP = 2
RCAP = 1024
RPT = RCAP // NS

_COL = np.arange(HO)
_PERM = (_COL % 8) * OUT + 2 * (_COL // 16) + (_COL % 16) // 8


def _build_attn_mat(attn):
    o_of_c = 2 * (_COL // 16) + (_COL % 16) // 8
    h_of_c = _COL % 8
    lane = np.arange(L)
    sel = jnp.asarray((h_of_c[:, None] == (lane[None, :] % 8)).astype(np.float32))
    vals = attn[:, h_of_c, o_of_c]
    return vals[:, :, None] * sel[None]


def _k1_body(h_ref, w_ref, al_ref, ar_ref, z_ref, el_ref, er_ref):
    z = jnp.dot(h_ref[...], w_ref[0], preferred_element_type=jnp.float32)
    z_ref[...] = z
    el_ref[...] = jnp.dot(z, al_ref[0], preferred_element_type=jnp.float32)
    er_ref[...] = jnp.dot(z, ar_ref[0], preferred_element_type=jnp.float32)


def _k1(h, W_perm, Al, Ar):
    nb = N // ROWS_TC
    return pl.pallas_call(
        _k1_body,
        grid=(M, nb),
        in_specs=[
            pl.BlockSpec((ROWS_TC, IN), lambda m, i: (i, 0)),
            pl.BlockSpec((1, IN, HO), lambda m, i: (m, 0, 0)),
            pl.BlockSpec((1, HO, L), lambda m, i: (m, 0, 0)),
            pl.BlockSpec((1, HO, L), lambda m, i: (m, 0, 0)),
        ],
        out_specs=[
            pl.BlockSpec((ROWS_TC, HO), lambda m, i: (m * nb + i, 0)),
            pl.BlockSpec((ROWS_TC, L), lambda m, i: (m * nb + i, 0)),
            pl.BlockSpec((ROWS_TC, L), lambda m, i: (m * nb + i, 0)),
        ],
        out_shape=[
            jax.ShapeDtypeStruct((M * N, HO), jnp.float32),
            jax.ShapeDtypeStruct((M * N, L), jnp.float32),
            jax.ShapeDtypeStruct((M * N, L), jnp.float32),
        ],
    )(h, W_perm, Al, Ar)


def _build_rank(bids_hbm, rank_v, bid_v):
    def zero(i, _):
        rank_v[pl.ds(i * L, L)] = jnp.zeros((L,), jnp.int32)
        return 0

    lax.fori_loop(0, N // L, zero, 0, unroll=False)

    pltpu.sync_copy(bids_hbm, bid_v.at[pl.ds(0, B)])
    ones = jnp.ones((L,), jnp.int32)

    def scat(i, _):
        idx = bid_v[pl.ds(i * L, L)]
        plsc.store_scatter(rank_v, [idx], ones)
        return 0

    lax.fori_loop(0, B // L, scat, 0, unroll=False)

    def ranks(i, cnt):
        v = rank_v[pl.ds(i * L, L)]
        cs = plsc.cumsum(v)
        r16 = jnp.where(v > 0, cnt + cs - 1, -1)
        rank_v[pl.ds(i * L, L)] = r16
        return cnt + jnp.sum(v)

    lax.fori_loop(0, N // L, ranks, jnp.int32(0), unroll=False)


def _k3_body(z_hbm, el_hbm, er_hbm, eisrc_hbm, eidst_hbm, bids_hbm,
             acc_out, s_out,
             rank_v, src_v, dst_v, psrc, prow, pdst,
             zbuf0, zbuf1, elbuf0, elbuf1, erbuf0, erbuf1,
             exbuf0, exbuf1, prow2d0, prow2d1,
             acc_s, s_s, semg0, semg1, sems0, sems1):
    c = lax.axis_index("c")
    s = lax.axis_index("s")

    _build_rank(bids_hbm, rank_v, src_v)

    zf = jnp.zeros((L,), jnp.float32)
    zi = jnp.zeros((L,), jnp.int32)

    def prefill(i, _):
        psrc[pl.ds(i * L, L)] = zi
        prow[pl.ds(i * L, L)] = zi
        pdst[pl.ds(i * L, L)] = zi
        return 0

    lax.fori_loop(0, PCAP // L, prefill, 0, unroll=False)

    zbufs = (zbuf0, zbuf1)
    elbufs = (elbuf0, elbuf1)
    erbufs = (erbuf0, erbuf1)
    exbufs = (exbuf0, exbuf1)
    prow2ds = (prow2d0, prow2d1)
    semgs = (semg0, semg1)
    semss = (sems0, sems1)

    for m in range(M):
        for p in range(NP):
            grp = c * NP + p
            def zb_zero(i, _):
                r = i // (HO // L)
                q = i % (HO // L)
                zbuf0[r, pl.ds(q * L, L)] = zf
                return 0

            lax.fori_loop(0, CK * (HO // L), zb_zero, 0, unroll=False)

            def ex_zero(i, _):
                exbuf0[i, pl.ds(0, L)] = zf
                return 0

            lax.fori_loop(0, CK, ex_zero, 0, unroll=False)

            pltpu.sync_copy(zbuf0, acc_s.at[pl.ds(s * RPT, RPT)])
            pltpu.sync_copy(exbuf0, s_s.at[pl.ds(s * RPT, RPT)])
            plsc.subcore_barrier()

            def seg_body(seg, _):
                ebase = m * E + s * EPT + seg * SEG
                pltpu.sync_copy(eisrc_hbm.at[pl.ds(ebase, SEG)],
                                src_v.at[pl.ds(0, SEG)])
                pltpu.sync_copy(eidst_hbm.at[pl.ds(ebase, SEG)],
                                dst_v.at[pl.ds(0, SEG)])

                def scan_body(g, mc):
                    d16 = dst_v[pl.ds(g * L, L)]
                    r16 = plsc.load_gather(rank_v, [d16])
                    matched = jnp.logical_and(r16 >= 0, (r16 & 3) == grp)
                    row16 = jnp.right_shift(r16, 2)
                    s16 = src_v[pl.ds(g * L, L)] + m * N
                    plsc.store_compressed(psrc.at[pl.ds(mc, L)], s16, mask=matched)
                    plsc.store_compressed(prow.at[pl.ds(mc, L)], row16, mask=matched)
                    plsc.store_compressed(pdst.at[pl.ds(mc, L)], d16 + m * N, mask=matched)
                    return mc + jnp.sum(matched.astype(jnp.int32))

                mc = lax.fori_loop(0, SEG // L, scan_body, jnp.int32(0), unroll=False)
                npair = (mc + 2 * CK - 1) // (2 * CK)

                def pair_body(j2, _):
                    bases = (j2 * 2 * CK, j2 * 2 * CK + CK)
                    gd = []
                    for b in range(2):
                        for jj in range(CK // L):
                            prow2ds[b][0, pl.ds(jj * L, L)] = \
                                prow[pl.ds(bases[b] + jj * L, L)]
                        gd.append((
                            pltpu.async_copy(
                                z_hbm.at[psrc.at[pl.ds(bases[b], CK)]],
                                zbufs[b], semgs[b]),
                            pltpu.async_copy(
                                el_hbm.at[psrc.at[pl.ds(bases[b], CK)]],
                                elbufs[b], semgs[b]),
                            pltpu.async_copy(
                                er_hbm.at[pdst.at[pl.ds(bases[b], CK)]],
                                erbufs[b], semgs[b]),
                        ))
                    sd = []
                    for b in range(2):
                        for d in gd[b]:
                            d.wait()
                        base = bases[b]
                        zbuf, elbuf, erbuf, exbuf = (zbufs[b], elbufs[b],
                                                     erbufs[b], exbufs[b])

                        def edge_body(e, _):
                            va = elbuf[e, pl.ds(0, L)]
                            vb = erbuf[e, pl.ds(0, L)]
                            t = va + vb
                            t = jnp.where(t >= 0, t, 0.2 * t)
                            ex = jnp.exp(t)
                            scale = jnp.where(base + e < mc, 1.0, 0.0)
                            ex = ex * scale.astype(jnp.float32)
                            exbuf[e, pl.ds(0, L)] = ex
                            for q in range(HO // L):
                                zbuf[e, pl.ds(q * L, L)] = \
                                    zbuf[e, pl.ds(q * L, L)] * ex
                            return 0

                        lax.fori_loop(0, CK, edge_body, 0, unroll=False)
                        sd.append(pltpu.async_copy(
                            zbufs[b], acc_s.at[prow2ds[b].at[0]], semss[b],
                            add=True))
                        sd.append(pltpu.async_copy(
                            exbufs[b], s_s.at[prow2ds[b].at[0]], semss[b],
                            add=True))
                    for d in sd:
                        d.wait()
                    return 0

                lax.fori_loop(0, npair, pair_body, 0, unroll=False)
                return 0

            lax.fori_loop(0, NSEG, seg_body, 0, unroll=False)
            plsc.subcore_barrier()

            obase = (m * 4 + grp) * RCAP + s * RPT
            pltpu.sync_copy(acc_s.at[pl.ds(s * RPT, RPT)],
                            acc_out.at[pl.ds(obase, RPT)])
            pltpu.sync_copy(s_s.at[pl.ds(s * RPT, RPT)],
                            s_out.at[pl.ds(obase, RPT)])
            plsc.subcore_barrier()


def _k3(z_hbm, el_hbm, er_hbm, ei_src, ei_dst, b_ids):
    mesh = plsc.VectorSubcoreMesh(core_axis_name="c", subcore_axis_name="s")
    f = pl.kernel(
        _k3_body,
        out_type=[
            jax.ShapeDtypeStruct((M * 2 * NP * RCAP, HO), jnp.float32),
            jax.ShapeDtypeStruct((M * 2 * NP * RCAP, L), jnp.float32),
        ],
        mesh=mesh,
        scratch_types=[
            pltpu.VMEM((N,), jnp.int32),
            pltpu.VMEM((B,), jnp.int32),
            pltpu.VMEM((B,), jnp.int32),
            pltpu.VMEM((PCAP,), jnp.int32),
            pltpu.VMEM((PCAP,), jnp.int32),
            pltpu.VMEM((PCAP,), jnp.int32),
            pltpu.VMEM((CK, HO), jnp.float32),
            pltpu.VMEM((CK, HO), jnp.float32),
            pltpu.VMEM((CK, L), jnp.float32),
            pltpu.VMEM((CK, L), jnp.float32),
            pltpu.VMEM((CK, L), jnp.float32),
            pltpu.VMEM((CK, L), jnp.float32),
            pltpu.VMEM((CK, L), jnp.float32),
            pltpu.VMEM((CK, L), jnp.float32),
            pltpu.VMEM((1, CK), jnp.int32),
            pltpu.VMEM((1, CK), jnp.int32),
            pltpu.VMEM_SHARED((RCAP, HO), jnp.float32),
            pltpu.VMEM_SHARED((RCAP, L), jnp.float32),
            pltpu.SemaphoreType.DMA,
            pltpu.SemaphoreType.DMA,
            pltpu.SemaphoreType.DMA,
            pltpu.SemaphoreType.DMA,
        ],
        compiler_params=pltpu.CompilerParams(needs_layout_passes=False, use_tc_tiling_on_sc=False),
    )
    return f(z_hbm, el_hbm, er_hbm, ei_src, ei_dst, b_ids)


def _k4_body(acc_hbm, s_hbm, bids_hbm, zb_out, sb_out,
             rank_v, bid_v, idx_v, zrows, srows, sem):
    c = lax.axis_index("c")
    s = lax.axis_index("s")
    wid = s * NC + c
    bpt = B // (NC * NS)

    _build_rank(bids_hbm, rank_v, bid_v)

    for m in range(M):
        def mk_idx(g, _):
            b16 = bid_v[pl.ds(wid * bpt + g * L, L)]
            r16 = plsc.load_gather(rank_v, [b16])
            fi = (m * 4 + (r16 & 3)) * RCAP + jnp.right_shift(r16, 2)
            idx_v[pl.ds(g * L, L)] = fi
            return 0

        lax.fori_loop(0, bpt // L, mk_idx, 0, unroll=False)
        pltpu.async_copy(acc_hbm.at[idx_v], zrows, sem).wait()
        pltpu.async_copy(s_hbm.at[idx_v], srows, sem).wait()
        obase = m * B + wid * bpt
        pltpu.sync_copy(zrows, zb_out.at[pl.ds(obase, bpt)])
        pltpu.sync_copy(srows, sb_out.at[pl.ds(obase, bpt)])


def _k4(acc, sacc, b_ids):
    mesh = plsc.VectorSubcoreMesh(core_axis_name="c", subcore_axis_name="s")
    bpt = B // (NC * NS)
    f = pl.kernel(
        _k4_body,
        out_type=[
            jax.ShapeDtypeStruct((M * B, HO), jnp.float32),
            jax.ShapeDtypeStruct((M * B, L), jnp.float32),
        ],
        mesh=mesh,
        scratch_types=[
            pltpu.VMEM((N,), jnp.int32),
            pltpu.VMEM((B,), jnp.int32),
            pltpu.VMEM((bpt,), jnp.int32),
            pltpu.VMEM((bpt, HO), jnp.float32),
            pltpu.VMEM((bpt, L), jnp.float32),
            pltpu.SemaphoreType.DMA,
        ],
        compiler_params=pltpu.CompilerParams(needs_layout_passes=False, use_tc_tiling_on_sc=False),
    )
    return f(acc, sacc, b_ids)


def _k5_body(zb_ref, sb_ref, biasP_ref, rp_ref, sw1_ref, sb1_ref, sw2_ref,
             wpart_ref):
    m = pl.program_id(0)
    sel = (lax.broadcasted_iota(jnp.int32, (M, 1), 0) == m).astype(jnp.float32)
    bias_row = jnp.sum(biasP_ref[...] * sel, axis=0, keepdims=True)
    den = jnp.dot(sb_ref[0], rp_ref[...],
                  preferred_element_type=jnp.float32) + 1e-9
    embp = zb_ref[0] / den + bias_row
    t = jnp.tanh(jnp.dot(embp, sw1_ref[...],
                         preferred_element_type=jnp.float32) + sb1_ref[...][None, :])
    w = jnp.dot(t, sw2_ref[...], preferred_element_type=jnp.float32)
    wpart_ref[...] = jnp.sum(w).reshape(1, 1, 1, 1)


def _k5(zb3, sb3, biasP, Rp16, sw1p, sa_b1, sa_w2):
    nb = B // 512
    return pl.pallas_call(
        _k5_body,
        grid=(M, nb),
        in_specs=[
            pl.BlockSpec((1, 512, HO), lambda m, i: (m, i, 0)),
            pl.BlockSpec((1, 512, L), lambda m, i: (m, i, 0)),
            pl.BlockSpec((M, HO), lambda m, i: (0, 0)),
            pl.BlockSpec((L, HO), lambda m, i: (0, 0)),
            pl.BlockSpec((HO, 64), lambda m, i: (0, 0)),
            pl.BlockSpec((64,), lambda m, i: (0,)),
            pl.BlockSpec((64, 1), lambda m, i: (0, 0)),
        ],
        out_specs=pl.BlockSpec((1, 1, 1, 1), lambda m, i: (m, i, 0, 0)),
        out_shape=jax.ShapeDtypeStruct((M, nb, 1, 1), jnp.float32),
    )(zb3, sb3, biasP, Rp16, sw1p, sa_b1, sa_w2)


def _k6_body(wpart_ref, zb_ref, sb_ref, biasP_ref, rp_ref, pinv_ref, out_ref):
    w0 = jnp.sum(wpart_ref[0]) / B
    w1 = jnp.sum(wpart_ref[1]) / B
    w2 = jnp.sum(wpart_ref[2]) / B
    mx = jnp.maximum(jnp.maximum(w0, w1), w2)
    e0 = jnp.exp(w0 - mx)
    e1 = jnp.exp(w1 - mx)
    e2 = jnp.exp(w2 - mx)
    tot = e0 + e1 + e2
    betas = (e0 / tot, e1 / tot, e2 / tot)
    acc = jnp.zeros((512, HO), jnp.float32)
    for m in range(M):
        den = jnp.dot(sb_ref[m], rp_ref[...],
                      preferred_element_type=jnp.float32) + 1e-9
        embp = zb_ref[m] / den + biasP_ref[m][None, :]
        acc = acc + betas[m] * embp
    out_ref[...] = jnp.dot(acc, pinv_ref[...], preferred_element_type=jnp.float32)


def _k6(wpart, zb3, sb3, biasP, Rp16, Pinv):
    nb = B // 512
    return pl.pallas_call(
        _k6_body,
        grid=(nb,),
        in_specs=[
            pl.BlockSpec((M, nb, 1, 1), lambda i: (0, 0, 0, 0)),
            pl.BlockSpec((M, 512, HO), lambda i: (0, i, 0)),
            pl.BlockSpec((M, 512, L), lambda i: (0, i, 0)),
            pl.BlockSpec((M, HO), lambda i: (0, 0)),
            pl.BlockSpec((L, HO), lambda i: (0, 0)),
            pl.BlockSpec((HO, HO), lambda i: (0, 0)),
        ],
        out_specs=pl.BlockSpec((512, HO), lambda i: (i, 0)),
        out_shape=jax.ShapeDtypeStruct((B, HO), jnp.float32),
    )(wpart, zb3, sb3, biasP, Rp16, Pinv)


def kernel(h, edge_index, b_ids, W_gat, attn_l, attn_r, gat_bias, sa_w1, sa_b1, sa_w2):
    perm = _PERM
    W_perm = W_gat[:, :, perm]
    Al = _build_attn_mat(attn_l)
    Ar = _build_attn_mat(attn_r)

    z_hbm, el_hbm, er_hbm = _k1(h, W_perm, Al, Ar)

    ei_src = edge_index[:, 0, :].reshape(M * E)
    ei_dst = edge_index[:, 1, :].reshape(M * E)
    acc, sacc = _k3(z_hbm, el_hbm, er_hbm, ei_src, ei_dst, b_ids)
    zb, sb = _k4(acc, sacc, b_ids)

    zb3 = zb.reshape(M, B, HO)
    sb3 = sb.reshape(M, B, L)
    biasP = gat_bias[:, perm]
    lane = np.arange(L)
    Rp16 = jnp.asarray(((lane[:, None] % 8 == _COL[None, :] % 8)
                        & (lane[:, None] < 8)).astype(np.float32))
    sw1p = sa_w1[perm]
    Pinv = jnp.asarray(np.eye(HO, dtype=np.float32)[_PERM])

    wpart = _k5(zb3, sb3, biasP, Rp16, sw1p, sa_b1, sa_w2)
    return _k6(wpart, zb3, sb3, biasP, Rp16, Pinv)


# speedup vs baseline: 2.6507x; 2.6507x over previous
import functools

import jax
import jax.numpy as jnp
import numpy as np
from jax import lax
from jax.experimental import pallas as pl
from jax.experimental.pallas import tpu as pltpu
from jax.experimental.pallas import tpu_sc as plsc

N = 10000
E = 320000
M = 3
IN = 128
H = 8
OUT = 64
HO = H * OUT
B = 4096

NC = 2
NS = 16
L = 16

ROWS_TC = 1000
EPT = E // NS
SEG = 4000
NSEG = EPT // SEG
PCAP = SEG + 64
CK = 64
NP = 2
RCAP = 1024
RPT = RCAP // NS

_COL = np.arange(HO)
_PERM = (_COL % 8) * OUT + 2 * (_COL // 16) + (_COL % 16) // 8


def _build_attn_mat(attn):
    o_of_c = 2 * (_COL // 16) + (_COL % 16) // 8
    h_of_c = _COL % 8
    lane = np.arange(L)
    sel = jnp.asarray((h_of_c[:, None] == (lane[None, :] % 8)).astype(np.float32))
    vals = attn[:, h_of_c, o_of_c]
    return vals[:, :, None] * sel[None]


def _k1_body(h_ref, w_ref, al_ref, ar_ref, z_ref, el_ref, er_ref):
    z = jnp.dot(h_ref[...], w_ref[0], preferred_element_type=jnp.float32)
    z_ref[...] = z
    el_ref[...] = jnp.dot(z, al_ref[0], preferred_element_type=jnp.float32)
    er_ref[...] = jnp.dot(z, ar_ref[0], preferred_element_type=jnp.float32)


def _k1(h, W_perm, Al, Ar):
    nb = N // ROWS_TC
    return pl.pallas_call(
        _k1_body,
        grid=(M, nb),
        in_specs=[
            pl.BlockSpec((ROWS_TC, IN), lambda m, i: (i, 0)),
            pl.BlockSpec((1, IN, HO), lambda m, i: (m, 0, 0)),
            pl.BlockSpec((1, HO, L), lambda m, i: (m, 0, 0)),
            pl.BlockSpec((1, HO, L), lambda m, i: (m, 0, 0)),
        ],
        out_specs=[
            pl.BlockSpec((ROWS_TC, HO), lambda m, i: (m * nb + i, 0)),
            pl.BlockSpec((ROWS_TC, L), lambda m, i: (m * nb + i, 0)),
            pl.BlockSpec((ROWS_TC, L), lambda m, i: (m * nb + i, 0)),
        ],
        out_shape=[
            jax.ShapeDtypeStruct((M * N, HO), jnp.float32),
            jax.ShapeDtypeStruct((M * N, L), jnp.float32),
            jax.ShapeDtypeStruct((M * N, L), jnp.float32),
        ],
    )(h, W_perm, Al, Ar)


def _build_rank(bids_hbm, rank_v, bid_v):
    def zero(i, _):
        rank_v[pl.ds(i * L, L)] = jnp.zeros((L,), jnp.int32)
        return 0

    lax.fori_loop(0, N // L, zero, 0, unroll=False)

    pltpu.sync_copy(bids_hbm, bid_v.at[pl.ds(0, B)])
    ones = jnp.ones((L,), jnp.int32)

    def scat(i, _):
        idx = bid_v[pl.ds(i * L, L)]
        plsc.store_scatter(rank_v, [idx], ones)
        return 0

    lax.fori_loop(0, B // L, scat, 0, unroll=False)

    def ranks(i, cnt):
        v = rank_v[pl.ds(i * L, L)]
        cs = plsc.cumsum(v)
        r16 = jnp.where(v > 0, cnt + cs - 1, -1)
        rank_v[pl.ds(i * L, L)] = r16
        return cnt + jnp.sum(v)

    lax.fori_loop(0, N // L, ranks, jnp.int32(0), unroll=False)


def _k3_body(z_hbm, el_hbm, er_hbm, eisrc_hbm, eidst_hbm, bids_hbm,
             acc_out, s_out,
             rank_v, src_v, dst_v, psrc, prow, pdst,
             zbuf, elbuf, erbuf, exbuf, prow2d, zero512, zero16,
             acc_s, s_s, sem):
    c = lax.axis_index("c")
    s = lax.axis_index("s")

    _build_rank(bids_hbm, rank_v, src_v)

    zf = jnp.zeros((L,), jnp.float32)
    zi = jnp.zeros((L,), jnp.int32)

    def prefill(i, _):
        psrc[pl.ds(i * L, L)] = zi
        prow[pl.ds(i * L, L)] = zi
        pdst[pl.ds(i * L, L)] = zi
        return 0

    lax.fori_loop(0, PCAP // L, prefill, 0, unroll=False)

    def z512(i, _):
        r = i // (HO // L)
        q = i % (HO // L)
        zero512[r, pl.ds(q * L, L)] = zf
        return 0

    lax.fori_loop(0, 16 * (HO // L), z512, 0, unroll=False)

    def z16(i, _):
        zero16[i, pl.ds(0, L)] = zf
        return 0

    lax.fori_loop(0, RPT, z16, 0, unroll=False)

    for m in range(M):
        for p in range(NP):
            grp = c * NP + p
            for zz in range(RPT // 16):
                pltpu.sync_copy(zero512, acc_s.at[pl.ds(s * RPT + zz * 16, 16)])
            pltpu.sync_copy(zero16, s_s.at[pl.ds(s * RPT, RPT)])
            plsc.subcore_barrier()

            def seg_body(seg, _):
                ebase = m * E + s * EPT + seg * SEG
                pltpu.sync_copy(eisrc_hbm.at[pl.ds(ebase, SEG)],
                                src_v.at[pl.ds(0, SEG)])
                pltpu.sync_copy(eidst_hbm.at[pl.ds(ebase, SEG)],
                                dst_v.at[pl.ds(0, SEG)])

                def scan_body(g, mc):
                    d16 = dst_v[pl.ds(g * L, L)]
                    r16 = plsc.load_gather(rank_v, [d16])
                    matched = jnp.logical_and(r16 >= 0, (r16 & 3) == grp)
                    row16 = jnp.right_shift(r16, 2)
                    s16 = src_v[pl.ds(g * L, L)] + m * N
                    plsc.store_compressed(psrc.at[pl.ds(mc, L)], s16, mask=matched)
                    plsc.store_compressed(prow.at[pl.ds(mc, L)], row16, mask=matched)
                    plsc.store_compressed(pdst.at[pl.ds(mc, L)], d16 + m * N, mask=matched)
                    return mc + jnp.sum(matched.astype(jnp.int32))

                mc = lax.fori_loop(0, SEG // L, scan_body, jnp.int32(0), unroll=2)
                nch = (mc + CK - 1) // CK

                def proc_body(j, _):
                    base = j * CK
                    for jj in range(CK // L):
                        prow2d[0, pl.ds(jj * L, L)] = prow[pl.ds(base + jj * L, L)]
                    pltpu.async_copy(z_hbm.at[psrc.at[pl.ds(base, CK)]], zbuf, sem).wait()
                    pltpu.async_copy(el_hbm.at[psrc.at[pl.ds(base, CK)]], elbuf, sem).wait()
                    pltpu.async_copy(er_hbm.at[pdst.at[pl.ds(base, CK)]], erbuf, sem).wait()

                    def edge_body(e, _):
                        va = elbuf[e, pl.ds(0, L)]
                        vb = erbuf[e, pl.ds(0, L)]
                        t = va + vb
                        t = jnp.where(t >= 0, t, 0.2 * t)
                        ex = jnp.exp(t)
                        scale = jnp.where(base + e < mc, 1.0, 0.0).astype(jnp.float32)
                        ex = ex * scale
                        exbuf[e, pl.ds(0, L)] = ex
                        for q in range(HO // L):
                            zbuf[e, pl.ds(q * L, L)] = zbuf[e, pl.ds(q * L, L)] * ex
                        return 0

                    lax.fori_loop(0, CK, edge_body, 0, unroll=4)
                    pltpu.sync_copy(zbuf, acc_s.at[prow2d.at[0]], add=True)
                    pltpu.sync_copy(exbuf, s_s.at[prow2d.at[0]], add=True)
                    return 0

                lax.fori_loop(0, nch, proc_body, 0, unroll=False)
                return 0

            lax.fori_loop(0, NSEG, seg_body, 0, unroll=False)
            plsc.subcore_barrier()

            obase = (m * 4 + grp) * RCAP + s * RPT
            pltpu.sync_copy(acc_s.at[pl.ds(s * RPT, RPT)],
                            acc_out.at[pl.ds(obase, RPT)])
            pltpu.sync_copy(s_s.at[pl.ds(s * RPT, RPT)],
                            s_out.at[pl.ds(obase, RPT)])
            plsc.subcore_barrier()


def _k3(z_hbm, el_hbm, er_hbm, ei_src, ei_dst, b_ids):
    mesh = plsc.VectorSubcoreMesh(core_axis_name="c", subcore_axis_name="s")
    f = pl.kernel(
        _k3_body,
        out_type=[
            jax.ShapeDtypeStruct((M * 2 * NP * RCAP, HO), jnp.float32),
            jax.ShapeDtypeStruct((M * 2 * NP * RCAP, L), jnp.float32),
        ],
        mesh=mesh,
        scratch_types=[
            pltpu.VMEM((N,), jnp.int32),
            pltpu.VMEM((B,), jnp.int32),
            pltpu.VMEM((B,), jnp.int32),
            pltpu.VMEM((PCAP,), jnp.int32),
            pltpu.VMEM((PCAP,), jnp.int32),
            pltpu.VMEM((PCAP,), jnp.int32),
            pltpu.VMEM((CK, HO), jnp.float32),
            pltpu.VMEM((CK, L), jnp.float32),
            pltpu.VMEM((CK, L), jnp.float32),
            pltpu.VMEM((CK, L), jnp.float32),
            pltpu.VMEM((1, CK), jnp.int32),
            pltpu.VMEM((16, HO), jnp.float32),
            pltpu.VMEM((RPT, L), jnp.float32),
            pltpu.VMEM_SHARED((RCAP, HO), jnp.float32),
            pltpu.VMEM_SHARED((RCAP, L), jnp.float32),
            pltpu.SemaphoreType.DMA,
        ],
        compiler_params=pltpu.CompilerParams(needs_layout_passes=False, use_tc_tiling_on_sc=False),
    )
    return f(z_hbm, el_hbm, er_hbm, ei_src, ei_dst, b_ids)


def _k4_body(acc_hbm, s_hbm, bids_hbm, zb_out, sb_out,
             rank_v, bid_v, idx_v, zrows, srows, sem):
    c = lax.axis_index("c")
    s = lax.axis_index("s")
    wid = s * NC + c
    bpt = B // (NC * NS)

    _build_rank(bids_hbm, rank_v, bid_v)

    for m in range(M):
        def mk_idx(g, _):
            b16 = bid_v[pl.ds(wid * bpt + g * L, L)]
            r16 = plsc.load_gather(rank_v, [b16])
            fi = (m * 4 + (r16 & 3)) * RCAP + jnp.right_shift(r16, 2)
            idx_v[pl.ds(g * L, L)] = fi
            return 0

        lax.fori_loop(0, bpt // L, mk_idx, 0, unroll=False)
        pltpu.async_copy(acc_hbm.at[idx_v], zrows, sem).wait()
        pltpu.async_copy(s_hbm.at[idx_v], srows, sem).wait()
        obase = m * B + wid * bpt
        pltpu.sync_copy(zrows, zb_out.at[pl.ds(obase, bpt)])
        pltpu.sync_copy(srows, sb_out.at[pl.ds(obase, bpt)])


def _k4(acc, sacc, b_ids):
    mesh = plsc.VectorSubcoreMesh(core_axis_name="c", subcore_axis_name="s")
    bpt = B // (NC * NS)
    f = pl.kernel(
        _k4_body,
        out_type=[
            jax.ShapeDtypeStruct((M * B, HO), jnp.float32),
            jax.ShapeDtypeStruct((M * B, L), jnp.float32),
        ],
        mesh=mesh,
        scratch_types=[
            pltpu.VMEM((N,), jnp.int32),
            pltpu.VMEM((B,), jnp.int32),
            pltpu.VMEM((bpt,), jnp.int32),
            pltpu.VMEM((bpt, HO), jnp.float32),
            pltpu.VMEM((bpt, L), jnp.float32),
            pltpu.SemaphoreType.DMA,
        ],
        compiler_params=pltpu.CompilerParams(needs_layout_passes=False, use_tc_tiling_on_sc=False),
    )
    return f(acc, sacc, b_ids)


def _k5_body(zb_ref, sb_ref, biasP_ref, rp_ref, sw1_ref, sb1_ref, sw2_ref,
             wpart_ref):
    m = pl.program_id(0)
    sel = (lax.broadcasted_iota(jnp.int32, (M, 1), 0) == m).astype(jnp.float32)
    bias_row = jnp.sum(biasP_ref[...] * sel, axis=0, keepdims=True)
    den = jnp.dot(sb_ref[0], rp_ref[...],
                  preferred_element_type=jnp.float32) + 1e-9
    embp = zb_ref[0] / den + bias_row
    t = jnp.tanh(jnp.dot(embp, sw1_ref[...],
                         preferred_element_type=jnp.float32) + sb1_ref[...][None, :])
    w = jnp.dot(t, sw2_ref[...], preferred_element_type=jnp.float32)
    wpart_ref[...] = jnp.sum(w).reshape(1, 1, 1, 1)


def _k5(zb3, sb3, biasP, Rp16, sw1p, sa_b1, sa_w2):
    nb = B // 512
    return pl.pallas_call(
        _k5_body,
        grid=(M, nb),
        in_specs=[
            pl.BlockSpec((1, 512, HO), lambda m, i: (m, i, 0)),
            pl.BlockSpec((1, 512, L), lambda m, i: (m, i, 0)),
            pl.BlockSpec((M, HO), lambda m, i: (0, 0)),
            pl.BlockSpec((L, HO), lambda m, i: (0, 0)),
            pl.BlockSpec((HO, 64), lambda m, i: (0, 0)),
            pl.BlockSpec((64,), lambda m, i: (0,)),
            pl.BlockSpec((64, 1), lambda m, i: (0, 0)),
        ],
        out_specs=pl.BlockSpec((1, 1, 1, 1), lambda m, i: (m, i, 0, 0)),
        out_shape=jax.ShapeDtypeStruct((M, nb, 1, 1), jnp.float32),
    )(zb3, sb3, biasP, Rp16, sw1p, sa_b1, sa_w2)


def _k6_body(wpart_ref, zb_ref, sb_ref, biasP_ref, rp_ref, pinv_ref, out_ref):
    w0 = jnp.sum(wpart_ref[0]) / B
    w1 = jnp.sum(wpart_ref[1]) / B
    w2 = jnp.sum(wpart_ref[2]) / B
    mx = jnp.maximum(jnp.maximum(w0, w1), w2)
    e0 = jnp.exp(w0 - mx)
    e1 = jnp.exp(w1 - mx)
    e2 = jnp.exp(w2 - mx)
    tot = e0 + e1 + e2
    betas = (e0 / tot, e1 / tot, e2 / tot)
    acc = jnp.zeros((512, HO), jnp.float32)
    for m in range(M):
        den = jnp.dot(sb_ref[m], rp_ref[...],
                      preferred_element_type=jnp.float32) + 1e-9
        embp = zb_ref[m] / den + biasP_ref[m][None, :]
        acc = acc + betas[m] * embp
    out_ref[...] = jnp.dot(acc, pinv_ref[...], preferred_element_type=jnp.float32)


def _k6(wpart, zb3, sb3, biasP, Rp16, Pinv):
    nb = B // 512
    return pl.pallas_call(
        _k6_body,
        grid=(nb,),
        in_specs=[
            pl.BlockSpec((M, nb, 1, 1), lambda i: (0, 0, 0, 0)),
            pl.BlockSpec((M, 512, HO), lambda i: (0, i, 0)),
            pl.BlockSpec((M, 512, L), lambda i: (0, i, 0)),
            pl.BlockSpec((M, HO), lambda i: (0, 0)),
            pl.BlockSpec((L, HO), lambda i: (0, 0)),
            pl.BlockSpec((HO, HO), lambda i: (0, 0)),
        ],
        out_specs=pl.BlockSpec((512, HO), lambda i: (i, 0)),
        out_shape=jax.ShapeDtypeStruct((B, HO), jnp.float32),
    )(wpart, zb3, sb3, biasP, Rp16, Pinv)


def kernel(h, edge_index, b_ids, W_gat, attn_l, attn_r, gat_bias, sa_w1, sa_b1, sa_w2):
    perm = _PERM
    W_perm = W_gat[:, :, perm]
    Al = _build_attn_mat(attn_l)
    Ar = _build_attn_mat(attn_r)

    z_hbm, el_hbm, er_hbm = _k1(h, W_perm, Al, Ar)

    ei_src = edge_index[:, 0, :].reshape(M * E)
    ei_dst = edge_index[:, 1, :].reshape(M * E)
    acc, sacc = _k3(z_hbm, el_hbm, er_hbm, ei_src, ei_dst, b_ids)
    zb, sb = _k4(acc, sacc, b_ids)

    zb3 = zb.reshape(M, B, HO)
    sb3 = sb.reshape(M, B, L)
    biasP = gat_bias[:, perm]
    lane = np.arange(L)
    Rp16 = jnp.asarray(((lane[:, None] % 8 == _COL[None, :] % 8)
                        & (lane[:, None] < 8)).astype(np.float32))
    sw1p = sa_w1[perm]
    Pinv = jnp.asarray(np.eye(HO, dtype=np.float32)[_PERM])

    wpart = _k5(zb3, sb3, biasP, Rp16, sw1p, sa_b1, sa_w2)
    return _k6(wpart, zb3, sb3, biasP, Rp16, Pinv)


# speedup vs baseline: 3.0546x; 1.1523x over previous
import functools

import jax
import jax.numpy as jnp
import numpy as np
from jax import lax
from jax.experimental import pallas as pl
from jax.experimental.pallas import tpu as pltpu
from jax.experimental.pallas import tpu_sc as plsc

N = 10000
E = 320000
M = 3
IN = 128
H = 8
OUT = 64
HO = H * OUT
B = 4096

NC = 2
NS = 16
L = 16

ROWS_TC = 1000
EPT = E // NS
SEG = 4000
NSEG = EPT // SEG
PCAP = SEG + 64
CK = 64
NP = 2
RCAP = 1024
RPT = RCAP // NS

_COL = np.arange(HO)
_PERM = (_COL % 8) * OUT + 2 * (_COL // 16) + (_COL % 16) // 8


def _build_attn_mat(attn):
    o_of_c = 2 * (_COL // 16) + (_COL % 16) // 8
    h_of_c = _COL % 8
    lane = np.arange(L)
    sel = jnp.asarray((h_of_c[:, None] == (lane[None, :] % 8)).astype(np.float32))
    vals = attn[:, h_of_c, o_of_c]
    return vals[:, :, None] * sel[None]


def _k1_body(h_ref, w_ref, al_ref, ar_ref, z_ref, el_ref, er_ref):
    z = jnp.dot(h_ref[...], w_ref[0], preferred_element_type=jnp.float32)
    z_ref[...] = z
    el_ref[...] = jnp.dot(z, al_ref[0], preferred_element_type=jnp.float32)
    er_ref[...] = jnp.dot(z, ar_ref[0], preferred_element_type=jnp.float32)


def _k1(h, W_perm, Al, Ar):
    nb = N // ROWS_TC
    return pl.pallas_call(
        _k1_body,
        grid=(M, nb),
        in_specs=[
            pl.BlockSpec((ROWS_TC, IN), lambda m, i: (i, 0)),
            pl.BlockSpec((1, IN, HO), lambda m, i: (m, 0, 0)),
            pl.BlockSpec((1, HO, L), lambda m, i: (m, 0, 0)),
            pl.BlockSpec((1, HO, L), lambda m, i: (m, 0, 0)),
        ],
        out_specs=[
            pl.BlockSpec((ROWS_TC, HO), lambda m, i: (m * nb + i, 0)),
            pl.BlockSpec((ROWS_TC, L), lambda m, i: (m * nb + i, 0)),
            pl.BlockSpec((ROWS_TC, L), lambda m, i: (m * nb + i, 0)),
        ],
        out_shape=[
            jax.ShapeDtypeStruct((M * N, HO), jnp.float32),
            jax.ShapeDtypeStruct((M * N, L), jnp.float32),
            jax.ShapeDtypeStruct((M * N, L), jnp.float32),
        ],
    )(h, W_perm, Al, Ar)


def _build_rank(bids_hbm, rank_v, bid_v):
    def zero(i, _):
        rank_v[pl.ds(i * L, L)] = jnp.zeros((L,), jnp.int32)
        return 0

    lax.fori_loop(0, N // L, zero, 0, unroll=False)

    pltpu.sync_copy(bids_hbm, bid_v.at[pl.ds(0, B)])
    ones = jnp.ones((L,), jnp.int32)

    def scat(i, _):
        idx = bid_v[pl.ds(i * L, L)]
        plsc.store_scatter(rank_v, [idx], ones)
        return 0

    lax.fori_loop(0, B // L, scat, 0, unroll=False)

    def ranks(i, cnt):
        v = rank_v[pl.ds(i * L, L)]
        cs = plsc.cumsum(v)
        r16 = jnp.where(v > 0, cnt + cs - 1, -1)
        rank_v[pl.ds(i * L, L)] = r16
        return cnt + jnp.sum(v)

    lax.fori_loop(0, N // L, ranks, jnp.int32(0), unroll=False)


def _k3_body(z_hbm, el_hbm, er_hbm, eisrc_hbm, eidst_hbm, bids_hbm,
             acc_out, s_out,
             rank_v, src_v, dst_v, psrc, prow, pdst,
             zbuf, elbuf, erbuf, exbuf, prow2d, zero512, zero16,
             acc_s, s_s, sem):
    c = lax.axis_index("c")
    s = lax.axis_index("s")

    _build_rank(bids_hbm, rank_v, src_v)

    zf = jnp.zeros((L,), jnp.float32)
    zi = jnp.zeros((L,), jnp.int32)

    def prefill(i, _):
        psrc[pl.ds(i * L, L)] = zi
        prow[pl.ds(i * L, L)] = zi
        pdst[pl.ds(i * L, L)] = zi
        return 0

    lax.fori_loop(0, PCAP // L, prefill, 0, unroll=False)

    def z512(i, _):
        r = i // (HO // L)
        q = i % (HO // L)
        zero512[r, pl.ds(q * L, L)] = zf
        return 0

    lax.fori_loop(0, 16 * (HO // L), z512, 0, unroll=False)

    def z16(i, _):
        zero16[i, pl.ds(0, L)] = zf
        return 0

    lax.fori_loop(0, RPT, z16, 0, unroll=False)

    for m in range(M):
        for p in range(NP):
            grp = c * NP + p
            for zz in range(RPT // 16):
                pltpu.sync_copy(zero512, acc_s.at[pl.ds(s * RPT + zz * 16, 16)])
            pltpu.sync_copy(zero16, s_s.at[pl.ds(s * RPT, RPT)])
            plsc.subcore_barrier()

            def seg_body(seg, _):
                ebase = m * E + s * EPT + seg * SEG
                d1 = pltpu.async_copy(eisrc_hbm.at[pl.ds(ebase, SEG)],
                                      src_v.at[pl.ds(0, SEG)], sem)
                d2 = pltpu.async_copy(eidst_hbm.at[pl.ds(ebase, SEG)],
                                      dst_v.at[pl.ds(0, SEG)], sem)
                d1.wait()
                d2.wait()

                def scan_body(g, mc):
                    d16 = dst_v[pl.ds(g * L, L)]
                    r16 = plsc.load_gather(rank_v, [d16])
                    matched = jnp.logical_and(r16 >= 0, (r16 & 3) == grp)
                    row16 = jnp.right_shift(r16, 2)
                    s16 = src_v[pl.ds(g * L, L)] + m * N
                    plsc.store_compressed(psrc.at[pl.ds(mc, L)], s16, mask=matched)
                    plsc.store_compressed(prow.at[pl.ds(mc, L)], row16, mask=matched)
                    plsc.store_compressed(pdst.at[pl.ds(mc, L)], d16 + m * N, mask=matched)
                    return mc + jnp.sum(matched.astype(jnp.int32))

                mc = lax.fori_loop(0, SEG // L, scan_body, jnp.int32(0), unroll=2)
                nch = (mc + CK - 1) // CK

                def proc_body(j, _):
                    base = j * CK
                    for jj in range(CK // L):
                        prow2d[0, pl.ds(jj * L, L)] = prow[pl.ds(base + jj * L, L)]
                    g1 = pltpu.async_copy(z_hbm.at[psrc.at[pl.ds(base, CK)]],
                                          zbuf, sem)
                    g2 = pltpu.async_copy(el_hbm.at[psrc.at[pl.ds(base, CK)]],
                                          elbuf, sem)
                    g3 = pltpu.async_copy(er_hbm.at[pdst.at[pl.ds(base, CK)]],
                                          erbuf, sem)
                    g1.wait()
                    g2.wait()
                    g3.wait()

                    def edge_body(e, _):
                        va = elbuf[e, pl.ds(0, L)]
                        vb = erbuf[e, pl.ds(0, L)]
                        t = va + vb
                        t = jnp.where(t >= 0, t, 0.2 * t)
                        ex = jnp.exp(t)
                        scale = jnp.where(base + e < mc, 1.0, 0.0).astype(jnp.float32)
                        ex = ex * scale
                        exbuf[e, pl.ds(0, L)] = ex
                        for q in range(HO // L):
                            zbuf[e, pl.ds(q * L, L)] = zbuf[e, pl.ds(q * L, L)] * ex
                        return 0

                    lax.fori_loop(0, CK, edge_body, 0, unroll=4)
                    s1 = pltpu.async_copy(zbuf, acc_s.at[prow2d.at[0]], sem,
                                          add=True)
                    s2 = pltpu.async_copy(exbuf, s_s.at[prow2d.at[0]], sem,
                                          add=True)
                    s1.wait()
                    s2.wait()
                    return 0

                lax.fori_loop(0, nch, proc_body, 0, unroll=False)
                return 0

            lax.fori_loop(0, NSEG, seg_body, 0, unroll=False)
            plsc.subcore_barrier()

            obase = (m * 4 + grp) * RCAP + s * RPT
            pltpu.sync_copy(acc_s.at[pl.ds(s * RPT, RPT)],
                            acc_out.at[pl.ds(obase, RPT)])
            pltpu.sync_copy(s_s.at[pl.ds(s * RPT, RPT)],
                            s_out.at[pl.ds(obase, RPT)])
            plsc.subcore_barrier()


def _k3(z_hbm, el_hbm, er_hbm, ei_src, ei_dst, b_ids):
    mesh = plsc.VectorSubcoreMesh(core_axis_name="c", subcore_axis_name="s")
    f = pl.kernel(
        _k3_body,
        out_type=[
            jax.ShapeDtypeStruct((M * 2 * NP * RCAP, HO), jnp.float32),
            jax.ShapeDtypeStruct((M * 2 * NP * RCAP, L), jnp.float32),
        ],
        mesh=mesh,
        scratch_types=[
            pltpu.VMEM((N,), jnp.int32),
            pltpu.VMEM((B,), jnp.int32),
            pltpu.VMEM((B,), jnp.int32),
            pltpu.VMEM((PCAP,), jnp.int32),
            pltpu.VMEM((PCAP,), jnp.int32),
            pltpu.VMEM((PCAP,), jnp.int32),
            pltpu.VMEM((CK, HO), jnp.float32),
            pltpu.VMEM((CK, L), jnp.float32),
            pltpu.VMEM((CK, L), jnp.float32),
            pltpu.VMEM((CK, L), jnp.float32),
            pltpu.VMEM((1, CK), jnp.int32),
            pltpu.VMEM((16, HO), jnp.float32),
            pltpu.VMEM((RPT, L), jnp.float32),
            pltpu.VMEM_SHARED((RCAP, HO), jnp.float32),
            pltpu.VMEM_SHARED((RCAP, L), jnp.float32),
            pltpu.SemaphoreType.DMA,
        ],
        compiler_params=pltpu.CompilerParams(needs_layout_passes=False, use_tc_tiling_on_sc=False),
    )
    return f(z_hbm, el_hbm, er_hbm, ei_src, ei_dst, b_ids)


def _k4_body(acc_hbm, s_hbm, bids_hbm, zb_out, sb_out,
             rank_v, bid_v, idx_v, zrows, srows, sem):
    c = lax.axis_index("c")
    s = lax.axis_index("s")
    wid = s * NC + c
    bpt = B // (NC * NS)

    _build_rank(bids_hbm, rank_v, bid_v)

    for m in range(M):
        def mk_idx(g, _):
            b16 = bid_v[pl.ds(wid * bpt + g * L, L)]
            r16 = plsc.load_gather(rank_v, [b16])
            fi = (m * 4 + (r16 & 3)) * RCAP + jnp.right_shift(r16, 2)
            idx_v[pl.ds(g * L, L)] = fi
            return 0

        lax.fori_loop(0, bpt // L, mk_idx, 0, unroll=False)
        pltpu.async_copy(acc_hbm.at[idx_v], zrows, sem).wait()
        pltpu.async_copy(s_hbm.at[idx_v], srows, sem).wait()
        obase = m * B + wid * bpt
        pltpu.sync_copy(zrows, zb_out.at[pl.ds(obase, bpt)])
        pltpu.sync_copy(srows, sb_out.at[pl.ds(obase, bpt)])


def _k4(acc, sacc, b_ids):
    mesh = plsc.VectorSubcoreMesh(core_axis_name="c", subcore_axis_name="s")
    bpt = B // (NC * NS)
    f = pl.kernel(
        _k4_body,
        out_type=[
            jax.ShapeDtypeStruct((M * B, HO), jnp.float32),
            jax.ShapeDtypeStruct((M * B, L), jnp.float32),
        ],
        mesh=mesh,
        scratch_types=[
            pltpu.VMEM((N,), jnp.int32),
            pltpu.VMEM((B,), jnp.int32),
            pltpu.VMEM((bpt,), jnp.int32),
            pltpu.VMEM((bpt, HO), jnp.float32),
            pltpu.VMEM((bpt, L), jnp.float32),
            pltpu.SemaphoreType.DMA,
        ],
        compiler_params=pltpu.CompilerParams(needs_layout_passes=False, use_tc_tiling_on_sc=False),
    )
    return f(acc, sacc, b_ids)


def _k5_body(zb_ref, sb_ref, biasP_ref, rp_ref, sw1_ref, sb1_ref, sw2_ref,
             wpart_ref):
    m = pl.program_id(0)
    sel = (lax.broadcasted_iota(jnp.int32, (M, 1), 0) == m).astype(jnp.float32)
    bias_row = jnp.sum(biasP_ref[...] * sel, axis=0, keepdims=True)
    den = jnp.dot(sb_ref[0], rp_ref[...],
                  preferred_element_type=jnp.float32) + 1e-9
    embp = zb_ref[0] / den + bias_row
    t = jnp.tanh(jnp.dot(embp, sw1_ref[...],
                         preferred_element_type=jnp.float32) + sb1_ref[...][None, :])
    w = jnp.dot(t, sw2_ref[...], preferred_element_type=jnp.float32)
    wpart_ref[...] = jnp.sum(w).reshape(1, 1, 1, 1)


def _k5(zb3, sb3, biasP, Rp16, sw1p, sa_b1, sa_w2):
    nb = B // 512
    return pl.pallas_call(
        _k5_body,
        grid=(M, nb),
        in_specs=[
            pl.BlockSpec((1, 512, HO), lambda m, i: (m, i, 0)),
            pl.BlockSpec((1, 512, L), lambda m, i: (m, i, 0)),
            pl.BlockSpec((M, HO), lambda m, i: (0, 0)),
            pl.BlockSpec((L, HO), lambda m, i: (0, 0)),
            pl.BlockSpec((HO, 64), lambda m, i: (0, 0)),
            pl.BlockSpec((64,), lambda m, i: (0,)),
            pl.BlockSpec((64, 1), lambda m, i: (0, 0)),
        ],
        out_specs=pl.BlockSpec((1, 1, 1, 1), lambda m, i: (m, i, 0, 0)),
        out_shape=jax.ShapeDtypeStruct((M, nb, 1, 1), jnp.float32),
    )(zb3, sb3, biasP, Rp16, sw1p, sa_b1, sa_w2)


def _k6_body(wpart_ref, zb_ref, sb_ref, biasP_ref, rp_ref, pinv_ref, out_ref):
    w0 = jnp.sum(wpart_ref[0]) / B
    w1 = jnp.sum(wpart_ref[1]) / B
    w2 = jnp.sum(wpart_ref[2]) / B
    mx = jnp.maximum(jnp.maximum(w0, w1), w2)
    e0 = jnp.exp(w0 - mx)
    e1 = jnp.exp(w1 - mx)
    e2 = jnp.exp(w2 - mx)
    tot = e0 + e1 + e2
    betas = (e0 / tot, e1 / tot, e2 / tot)
    acc = jnp.zeros((512, HO), jnp.float32)
    for m in range(M):
        den = jnp.dot(sb_ref[m], rp_ref[...],
                      preferred_element_type=jnp.float32) + 1e-9
        embp = zb_ref[m] / den + biasP_ref[m][None, :]
        acc = acc + betas[m] * embp
    out_ref[...] = jnp.dot(acc, pinv_ref[...], preferred_element_type=jnp.float32)


def _k6(wpart, zb3, sb3, biasP, Rp16, Pinv):
    nb = B // 512
    return pl.pallas_call(
        _k6_body,
        grid=(nb,),
        in_specs=[
            pl.BlockSpec((M, nb, 1, 1), lambda i: (0, 0, 0, 0)),
            pl.BlockSpec((M, 512, HO), lambda i: (0, i, 0)),
            pl.BlockSpec((M, 512, L), lambda i: (0, i, 0)),
            pl.BlockSpec((M, HO), lambda i: (0, 0)),
            pl.BlockSpec((L, HO), lambda i: (0, 0)),
            pl.BlockSpec((HO, HO), lambda i: (0, 0)),
        ],
        out_specs=pl.BlockSpec((512, HO), lambda i: (i, 0)),
        out_shape=jax.ShapeDtypeStruct((B, HO), jnp.float32),
    )(wpart, zb3, sb3, biasP, Rp16, Pinv)


def kernel(h, edge_index, b_ids, W_gat, attn_l, attn_r, gat_bias, sa_w1, sa_b1, sa_w2):
    perm = _PERM
    W_perm = W_gat[:, :, perm]
    Al = _build_attn_mat(attn_l)
    Ar = _build_attn_mat(attn_r)

    z_hbm, el_hbm, er_hbm = _k1(h, W_perm, Al, Ar)

    ei_src = edge_index[:, 0, :].reshape(M * E)
    ei_dst = edge_index[:, 1, :].reshape(M * E)
    acc, sacc = _k3(z_hbm, el_hbm, er_hbm, ei_src, ei_dst, b_ids)
    zb, sb = _k4(acc, sacc, b_ids)

    zb3 = zb.reshape(M, B, HO)
    sb3 = sb.reshape(M, B, L)
    biasP = gat_bias[:, perm]
    lane = np.arange(L)
    Rp16 = jnp.asarray(((lane[:, None] % 8 == _COL[None, :] % 8)
                        & (lane[:, None] < 8)).astype(np.float32))
    sw1p = sa_w1[perm]
    Pinv = jnp.asarray(np.eye(HO, dtype=np.float32)[_PERM])

    wpart = _k5(zb3, sb3, biasP, Rp16, sw1p, sa_b1, sa_w2)
    return _k6(wpart, zb3, sb3, biasP, Rp16, Pinv)
